# Initial kernel scaffold; baseline (speedup 1.0000x reference)
import jax
import jax.numpy as jnp
from jax.experimental import pallas as pl


def kernel(node_feature, geo_encoding, edge_index_2rd, edx_jk, edx_ij, edge_whichface, att, same_face, W_first, b_first, W_rest, b_rest):
    raise NotImplementedError("write your pallas kernel here")


# speedup vs baseline: 3.2037x; 3.2037x over previous
import functools

import jax
import jax.numpy as jnp
from jax import lax
from jax.experimental import pallas as pl
from jax.experimental.pallas import tpu as pltpu
from jax.experimental.pallas import tpu_sc as plsc

N = 10000
E = 320000
H = 128
GEO = 16
DEPTH = 3

NC = 2
NS = 16
NW = NC * NS

ROWS = 3 * E
R_PER_W = ROWS // NW
G_CHUNK = 120
G_ITERS = R_PER_W // G_CHUNK

@functools.cache
def _sc_mesh():
    return plsc.VectorSubcoreMesh(core_axis_name="c", subcore_axis_name="s",
                                  num_cores=NC, num_subcores=NS)


@functools.cache
def _gather_kernel_fn():
    return pl.kernel(
        _gather_body,
        out_type=jax.ShapeDtypeStruct((ROWS, H), jnp.float32),
        mesh=_sc_mesh(),
        scratch_types=[
            pltpu.VMEM((R_PER_W,), jnp.int32),
            pltpu.VMEM((G_CHUNK, H), jnp.float32),
            pltpu.VMEM((G_CHUNK, H), jnp.float32),
            pltpu.SemaphoreType.DMA,
            pltpu.SemaphoreType.DMA,
            pltpu.SemaphoreType.DMA,
            pltpu.SemaphoreType.DMA,
        ],
    )


def _gather_body(table_hbm, idx_hbm, out_hbm, idx_v, buf0, buf1,
                 sg0, sg1, so0, so1):
    c = lax.axis_index("c")
    sid = lax.axis_index("s")
    wid = sid * NC + c
    base = wid * R_PER_W

    pltpu.sync_copy(idx_hbm.at[pl.ds(base, R_PER_W)], idx_v)

    bufs = (buf0, buf1)
    gsems = (sg0, sg1)
    osems = (so0, so1)

    def start_g(t, b):
        return pltpu.async_copy(
            table_hbm.at[idx_v.at[pl.ds(t * G_CHUNK, G_CHUNK)]],
            bufs[b], gsems[b])

    def start_o(t, b):
        return pltpu.async_copy(
            bufs[b], out_hbm.at[pl.ds(base + t * G_CHUNK, G_CHUNK)], osems[b])

    start_g(0, 0)
    start_g(1, 1)

    def body(o, carry):
        t0 = o * 2
        for b in range(2):
            t = t0 + b
            pltpu.make_async_copy(
                table_hbm.at[idx_v.at[pl.ds(t * G_CHUNK, G_CHUNK)]],
                bufs[b], gsems[b]).wait()
            start_o(t, b)
        for b in range(2):
            t = t0 + b
            pltpu.make_async_copy(
                bufs[b], out_hbm.at[pl.ds(base + t * G_CHUNK, G_CHUNK)],
                osems[b]).wait()

            @pl.when(t + 2 < G_ITERS)
            def _():
                start_g(t + 2, b)

        return carry

    lax.fori_loop(0, G_ITERS // 2, body, 0)


CH = E // 128
CH_PER_CORE = CH // NC
T_ITERS = (CH_PER_CORE + NS - 1) // NS
N_PER_TILE = N // NS


@functools.cache
def _scatter_kernel_fn():
    return pl.kernel(
        _scatter_body,
        out_type=jax.ShapeDtypeStruct((NC, N, H), jnp.float32),
        mesh=_sc_mesh(),
        scratch_types=[
            pltpu.VMEM((128,), jnp.int32),
            pltpu.VMEM((128, H), jnp.float32),
            pltpu.VMEM_SHARED((N, H), jnp.float32),
        ],
    )


def _scatter_body(upd_hbm, idx2_hbm, zeros_hbm, out_hbm, idx_v, upd_v, acc_sh):
    c = lax.axis_index("c")
    s = lax.axis_index("s")

    @pl.when(s == 0)
    def _():
        pltpu.sync_copy(zeros_hbm, acc_sh)

    plsc.subcore_barrier()

    def body(t, carry):
        r = t * NS + s

        @pl.when(r < CH_PER_CORE)
        def _():
            j = c * CH_PER_CORE + r
            pltpu.sync_copy(idx2_hbm.at[j], idx_v)
            pltpu.sync_copy(upd_hbm.at[pl.ds(j * 128, 128)], upd_v)
            pltpu.sync_copy(upd_v, acc_sh.at[idx_v], add=True)

        return carry

    lax.fori_loop(0, T_ITERS, body, 0)
    plsc.subcore_barrier()

    @pl.when(s < 10)
    def _():
        pltpu.sync_copy(acc_sh.at[pl.ds(s * 1000, 1000)],
                        out_hbm.at[c, pl.ds(s * 1000, 1000)])


BE = 1280
GRID = E // BE
SLAB = E // BE


def _mlp_body(gi, gj, gk, geo, sf, wi, wj, wk, wg, b1, wr, br, out):
    f = jnp.float32
    h = (jnp.dot(gi[...].astype(jnp.bfloat16), wi[...], preferred_element_type=f)
         + jnp.dot(gj[...].astype(jnp.bfloat16), wj[...], preferred_element_type=f)
         + jnp.dot(gk[...].astype(jnp.bfloat16), wk[...], preferred_element_type=f)
         + jnp.dot(geo[...].astype(jnp.bfloat16), wg[...], preferred_element_type=f))
    h = jnp.maximum(h + b1[...][None, :], 0.0)
    for l in range(DEPTH):
        h = jnp.dot(h.astype(jnp.bfloat16), wr[l],
                    preferred_element_type=f)
        h = jnp.maximum(h + br[l][None, :], 0.0)
    out[...] = jnp.where(sf[...] > 0.0, h[:, :H], h[:, H:])


def _mlp_call(gath, geo, sf, wi, wj, wk, wg, b1, wrbd, brc):
    full = lambda a: pl.BlockSpec(a.shape, lambda e: (0,) * a.ndim)
    return pl.pallas_call(
        _mlp_body,
        grid=(GRID,),
        in_specs=[
            pl.BlockSpec((BE, H), lambda e: (e, 0)),
            pl.BlockSpec((BE, H), lambda e: (e + SLAB, 0)),
            pl.BlockSpec((BE, H), lambda e: (e + 2 * SLAB, 0)),
            pl.BlockSpec((BE, GEO), lambda e: (e, 0)),
            pl.BlockSpec((BE, 1), lambda e: (e, 0)),
            full(wi), full(wj), full(wk), full(wg), full(b1),
            full(wrbd), full(brc),
        ],
        out_specs=pl.BlockSpec((BE, H), lambda e: (e, 0)),
        out_shape=jax.ShapeDtypeStruct((E, H), jnp.float32),
        compiler_params=pltpu.CompilerParams(
            dimension_semantics=("arbitrary",)),
    )(gath, gath, gath, geo, sf, wi, wj, wk, wg, b1, wrbd, brc)


def _add_body(p, out):
    out[...] = p[0] + p[1]


def _add_call(partials):
    return pl.pallas_call(
        _add_body,
        grid=(10,),
        in_specs=[pl.BlockSpec((2, N // 10, H), lambda e: (0, e, 0))],
        out_specs=pl.BlockSpec((N // 10, H), lambda e: (e, 0)),
        out_shape=jax.ShapeDtypeStruct((N, H), jnp.float32),
    )(partials)


def kernel(node_feature, geo_encoding, edge_index_2rd, edx_jk, edx_ij,
           edge_whichface, att, same_face, W_first, b_first, W_rest, b_rest):
    idx_all = edge_index_2rd.reshape(ROWS).astype(jnp.int32)
    sf = same_face.astype(jnp.float32).reshape(E, 1)

    wf_t = jnp.transpose(W_first, (0, 2, 1))
    wr_t = jnp.transpose(W_rest, (0, 1, 3, 2))
    a = att[:, 0]
    wr_t = wr_t.at[:, DEPTH - 1].multiply(a[:, None, None])
    br = b_rest.at[:, DEPTH - 1].multiply(a[:, None])

    wcat = jnp.concatenate([wf_t[0], wf_t[1]], axis=1)
    wi = wcat[:H].astype(jnp.bfloat16)
    wj = wcat[H:2 * H].astype(jnp.bfloat16)
    wk = wcat[2 * H:3 * H].astype(jnp.bfloat16)
    wg = wcat[3 * H:].astype(jnp.bfloat16)
    b1 = jnp.concatenate([b_first[0], b_first[1]])
    z = jnp.zeros((DEPTH, H, H), jnp.float32)
    wrbd = jnp.concatenate([
        jnp.concatenate([wr_t[0], z], axis=2),
        jnp.concatenate([z, wr_t[1]], axis=2),
    ], axis=1).astype(jnp.bfloat16)
    brc = jnp.concatenate([br[0], br[1]], axis=1)

    gath = _gather_kernel_fn()(node_feature, idx_all)
    edge_out = _mlp_call(gath, geo_encoding, sf, wi, wj, wk, wg, b1, wrbd, brc)

    idx2 = edge_index_2rd[0].reshape(CH, 128).astype(jnp.int32)
    zeros = jnp.zeros((N, H), jnp.float32)
    partials = _scatter_kernel_fn()(edge_out, idx2, zeros)
    return _add_call(partials)


# speedup vs baseline: 3.2978x; 1.0294x over previous
import functools

import jax
import jax.numpy as jnp
from jax import lax
from jax.experimental import pallas as pl
from jax.experimental.pallas import tpu as pltpu
from jax.experimental.pallas import tpu_sc as plsc

N = 10000
E = 320000
H = 128
GEO = 16
DEPTH = 3

NC = 2
NS = 16
NW = NC * NS

NCHUNK = 5
EC = E // NCHUNK

G_ROWS = 3 * EC
R_PER_W = G_ROWS // NW
G_CHUNK = 120
G_ITERS = R_PER_W // G_CHUNK


@functools.cache
def _sc_mesh():
    return plsc.VectorSubcoreMesh(core_axis_name="c", subcore_axis_name="s",
                                  num_cores=NC, num_subcores=NS)


@functools.cache
def _gather_kernel_fn():
    return pl.kernel(
        _gather_body,
        out_type=jax.ShapeDtypeStruct((G_ROWS, H), jnp.float32),
        mesh=_sc_mesh(),
        scratch_types=[
            pltpu.VMEM((R_PER_W,), jnp.int32),
            pltpu.VMEM((G_CHUNK, H), jnp.float32),
            pltpu.VMEM((G_CHUNK, H), jnp.float32),
            pltpu.SemaphoreType.DMA,
            pltpu.SemaphoreType.DMA,
            pltpu.SemaphoreType.DMA,
            pltpu.SemaphoreType.DMA,
        ],
    )


def _gather_body(table_hbm, idx_hbm, out_hbm, idx_v, buf0, buf1,
                 sg0, sg1, so0, so1):
    c = lax.axis_index("c")
    sid = lax.axis_index("s")
    wid = sid * NC + c
    base = wid * R_PER_W

    pltpu.sync_copy(idx_hbm.at[pl.ds(base, R_PER_W)], idx_v)

    bufs = (buf0, buf1)
    gsems = (sg0, sg1)
    osems = (so0, so1)

    def start_g(t, b):
        return pltpu.async_copy(
            table_hbm.at[idx_v.at[pl.ds(t * G_CHUNK, G_CHUNK)]],
            bufs[b], gsems[b])

    def start_o(t, b):
        return pltpu.async_copy(
            bufs[b], out_hbm.at[pl.ds(base + t * G_CHUNK, G_CHUNK)], osems[b])

    start_g(0, 0)
    start_g(1, 1)

    def body(o, carry):
        t0 = o * 2
        for b in range(2):
            t = t0 + b
            pltpu.make_async_copy(
                table_hbm.at[idx_v.at[pl.ds(t * G_CHUNK, G_CHUNK)]],
                bufs[b], gsems[b]).wait()
            start_o(t, b)
        for b in range(2):
            t = t0 + b
            pltpu.make_async_copy(
                bufs[b], out_hbm.at[pl.ds(base + t * G_CHUNK, G_CHUNK)],
                osems[b]).wait()

            @pl.when(t + 2 < G_ITERS)
            def _():
                start_g(t + 2, b)

        return carry

    lax.fori_loop(0, G_ITERS // 2, body, 0)


CH = EC // 128
CH_PER_CORE = CH // NC
T_ITERS = (CH_PER_CORE + NS - 1) // NS


@functools.cache
def _scatter_kernel_fn():
    return pl.kernel(
        _scatter_body,
        out_type=jax.ShapeDtypeStruct((NC, N, H), jnp.float32),
        mesh=_sc_mesh(),
        scratch_types=[
            pltpu.VMEM((128,), jnp.int32),
            pltpu.VMEM((128,), jnp.int32),
            pltpu.VMEM((128, H), jnp.float32),
            pltpu.VMEM((128, H), jnp.float32),
            pltpu.VMEM_SHARED((N, H), jnp.float32),
            pltpu.SemaphoreType.DMA,
            pltpu.SemaphoreType.DMA,
        ],
    )


def _scatter_body(upd_hbm, idx2_hbm, zeros_hbm, out_hbm,
                  idx0, idx1, upd0, upd1, acc_sh, sm0, sm1):
    c = lax.axis_index("c")
    s = lax.axis_index("s")

    @pl.when(s == 0)
    def _():
        pltpu.sync_copy(zeros_hbm, acc_sh)

    plsc.subcore_barrier()

    idxb = (idx0, idx1)
    updb = (upd0, upd1)
    sems = (sm0, sm1)

    def stage(t, b):
        j = c * CH_PER_CORE + t * NS + s
        pltpu.async_copy(idx2_hbm.at[j], idxb[b], sems[b])
        pltpu.async_copy(upd_hbm.at[pl.ds(j * 128, 128)], updb[b], sems[b])

    def wait_stage(t, b):
        j = c * CH_PER_CORE + t * NS + s
        pltpu.make_async_copy(idx2_hbm.at[j], idxb[b], sems[b]).wait()
        pltpu.make_async_copy(
            upd_hbm.at[pl.ds(j * 128, 128)], updb[b], sems[b]).wait()

    stage(0, 0)

    def body(o, carry):
        t0 = o * 2
        for b in range(2):
            t = t0 + b

            @pl.when(t * NS + s < CH_PER_CORE)
            def _():
                wait_stage(t, b)

            @pl.when((t + 1) * NS + s < CH_PER_CORE)
            def _():
                stage(t + 1, 1 - b)

            @pl.when(t * NS + s < CH_PER_CORE)
            def _():
                pltpu.sync_copy(updb[b], acc_sh.at[idxb[b]], add=True)

        return carry

    lax.fori_loop(0, T_ITERS // 2, body, 0)
    plsc.subcore_barrier()

    @pl.when(s < 10)
    def _():
        pltpu.sync_copy(acc_sh.at[pl.ds(s * 1000, 1000)],
                        out_hbm.at[c, pl.ds(s * 1000, 1000)])


BE = 1280
GRID = EC // BE
SLAB = EC // BE


def _mlp_body(gi, gj, gk, geo, sf, wi, wj, wk, wg, b1, wr, br, out):
    f = jnp.float32
    h = (jnp.dot(gi[...].astype(jnp.bfloat16), wi[...], preferred_element_type=f)
         + jnp.dot(gj[...].astype(jnp.bfloat16), wj[...], preferred_element_type=f)
         + jnp.dot(gk[...].astype(jnp.bfloat16), wk[...], preferred_element_type=f)
         + jnp.dot(geo[...].astype(jnp.bfloat16), wg[...], preferred_element_type=f))
    h = jnp.maximum(h + b1[...][None, :], 0.0)
    for l in range(DEPTH):
        h = jnp.dot(h.astype(jnp.bfloat16), wr[l],
                    preferred_element_type=f)
        h = jnp.maximum(h + br[l][None, :], 0.0)
    out[...] = jnp.where(sf[...] > 0.0, h[:, :H], h[:, H:])


def _mlp_call(gath, geo, sf, wi, wj, wk, wg, b1, wrbd, brc):
    full = lambda a: pl.BlockSpec(a.shape, lambda e: (0,) * a.ndim)
    return pl.pallas_call(
        _mlp_body,
        grid=(GRID,),
        in_specs=[
            pl.BlockSpec((BE, H), lambda e: (e, 0)),
            pl.BlockSpec((BE, H), lambda e: (e + SLAB, 0)),
            pl.BlockSpec((BE, H), lambda e: (e + 2 * SLAB, 0)),
            pl.BlockSpec((BE, GEO), lambda e: (e, 0)),
            pl.BlockSpec((BE, 1), lambda e: (e, 0)),
            full(wi), full(wj), full(wk), full(wg), full(b1),
            full(wrbd), full(brc),
        ],
        out_specs=pl.BlockSpec((BE, H), lambda e: (e, 0)),
        out_shape=jax.ShapeDtypeStruct((EC, H), jnp.float32),
        compiler_params=pltpu.CompilerParams(
            dimension_semantics=("arbitrary",)),
    )(gath, gath, gath, geo, sf, wi, wj, wk, wg, b1, wrbd, brc)


def _add_body(p, out):
    out[...] = jnp.sum(p[...], axis=(0, 1))


def _add_call(partials):
    npart = partials.shape[0]
    return pl.pallas_call(
        _add_body,
        grid=(10,),
        in_specs=[pl.BlockSpec((npart, NC, N // 10, H),
                               lambda e: (0, 0, e, 0))],
        out_specs=pl.BlockSpec((N // 10, H), lambda e: (e, 0)),
        out_shape=jax.ShapeDtypeStruct((N, H), jnp.float32),
    )(partials)


def kernel(node_feature, geo_encoding, edge_index_2rd, edx_jk, edx_ij,
           edge_whichface, att, same_face, W_first, b_first, W_rest, b_rest):
    sf = same_face.astype(jnp.float32).reshape(E, 1)

    wf_t = jnp.transpose(W_first, (0, 2, 1))
    wr_t = jnp.transpose(W_rest, (0, 1, 3, 2))
    a = att[:, 0]
    wr_t = wr_t.at[:, DEPTH - 1].multiply(a[:, None, None])
    br = b_rest.at[:, DEPTH - 1].multiply(a[:, None])

    wcat = jnp.concatenate([wf_t[0], wf_t[1]], axis=1)
    wi = wcat[:H].astype(jnp.bfloat16)
    wj = wcat[H:2 * H].astype(jnp.bfloat16)
    wk = wcat[2 * H:3 * H].astype(jnp.bfloat16)
    wg = wcat[3 * H:].astype(jnp.bfloat16)
    b1 = jnp.concatenate([b_first[0], b_first[1]])
    z = jnp.zeros((DEPTH, H, H), jnp.float32)
    wrbd = jnp.concatenate([
        jnp.concatenate([wr_t[0], z], axis=2),
        jnp.concatenate([z, wr_t[1]], axis=2),
    ], axis=1).astype(jnp.bfloat16)
    brc = jnp.concatenate([br[0], br[1]], axis=1)

    ei = edge_index_2rd.astype(jnp.int32)
    zeros = jnp.zeros((N, H), jnp.float32)
    gfn = _gather_kernel_fn()
    sfn = _scatter_kernel_fn()

    parts = []
    for cidx in range(NCHUNK):
        lo = cidx * EC
        idx_c = ei[:, lo:lo + EC].reshape(G_ROWS)
        gath_c = gfn(node_feature, idx_c)
        eo_c = _mlp_call(gath_c, geo_encoding[lo:lo + EC], sf[lo:lo + EC],
                         wi, wj, wk, wg, b1, wrbd, brc)
        idx2_c = ei[0, lo:lo + EC].reshape(CH, 128)
        parts.append(sfn(eo_c, idx2_c, zeros))

    return _add_call(jnp.stack(parts))


# speedup vs baseline: 3.2992x; 1.0004x over previous
import functools

import jax
import jax.numpy as jnp
from jax import lax
from jax.experimental import pallas as pl
from jax.experimental.pallas import tpu as pltpu
from jax.experimental.pallas import tpu_sc as plsc

N = 10000
E = 320000
H = 128
GEO = 16
DEPTH = 3

NC = 2
NS = 16
NW = NC * NS

NCHUNK = 5
EC = E // NCHUNK

G_ROWS = 3 * EC
R_PER_W = G_ROWS // NW
G_CHUNK = 120
G_ITERS = R_PER_W // G_CHUNK


@functools.cache
def _sc_mesh():
    return plsc.VectorSubcoreMesh(core_axis_name="c", subcore_axis_name="s",
                                  num_cores=NC, num_subcores=NS)


@functools.cache
def _gather_kernel_fn():
    return pl.kernel(
        _gather_body,
        out_type=jax.ShapeDtypeStruct((G_ROWS, H), jnp.float32),
        mesh=_sc_mesh(),
        scratch_types=[
            pltpu.VMEM((R_PER_W,), jnp.int32),
            pltpu.VMEM((G_CHUNK, H), jnp.float32),
            pltpu.VMEM((G_CHUNK, H), jnp.float32),
            pltpu.SemaphoreType.DMA,
            pltpu.SemaphoreType.DMA,
            pltpu.SemaphoreType.DMA,
            pltpu.SemaphoreType.DMA,
        ],
    )


def _gather_body(table_hbm, idx_hbm, out_hbm, idx_v, buf0, buf1,
                 sg0, sg1, so0, so1):
    c = lax.axis_index("c")
    sid = lax.axis_index("s")
    wid = sid * NC + c
    base = wid * R_PER_W

    pltpu.sync_copy(idx_hbm.at[pl.ds(base, R_PER_W)], idx_v)

    bufs = (buf0, buf1)
    gsems = (sg0, sg1)
    osems = (so0, so1)

    def start_g(t, b):
        return pltpu.async_copy(
            table_hbm.at[idx_v.at[pl.ds(t * G_CHUNK, G_CHUNK)]],
            bufs[b], gsems[b])

    def start_o(t, b):
        return pltpu.async_copy(
            bufs[b], out_hbm.at[pl.ds(base + t * G_CHUNK, G_CHUNK)], osems[b])

    start_g(0, 0)
    start_g(1, 1)

    def body(o, carry):
        t0 = o * 2
        for b in range(2):
            t = t0 + b
            pltpu.make_async_copy(
                table_hbm.at[idx_v.at[pl.ds(t * G_CHUNK, G_CHUNK)]],
                bufs[b], gsems[b]).wait()
            start_o(t, b)
        for b in range(2):
            t = t0 + b
            pltpu.make_async_copy(
                bufs[b], out_hbm.at[pl.ds(base + t * G_CHUNK, G_CHUNK)],
                osems[b]).wait()

            @pl.when(t + 2 < G_ITERS)
            def _():
                start_g(t + 2, b)

        return carry

    lax.fori_loop(0, G_ITERS // 2, body, 0)


CH = EC // 128
CH_PER_CORE = CH // NC
T_ITERS = (CH_PER_CORE + NS - 1) // NS


@functools.cache
def _scatter_kernel_fn():
    return pl.kernel(
        _scatter_body,
        out_type=jax.ShapeDtypeStruct((NC, N, H), jnp.float32),
        mesh=_sc_mesh(),
        scratch_types=[
            pltpu.VMEM((128,), jnp.int32),
            pltpu.VMEM((128,), jnp.int32),
            pltpu.VMEM((128, H), jnp.float32),
            pltpu.VMEM((128, H), jnp.float32),
            pltpu.VMEM_SHARED((N, H), jnp.float32),
            pltpu.SemaphoreType.DMA,
            pltpu.SemaphoreType.DMA,
        ],
    )


def _scatter_body(upd_hbm, idx2_hbm, zeros_hbm, out_hbm,
                  idx0, idx1, upd0, upd1, acc_sh, sm0, sm1):
    c = lax.axis_index("c")
    s = lax.axis_index("s")

    @pl.when(s == 0)
    def _():
        pltpu.sync_copy(zeros_hbm, acc_sh)

    plsc.subcore_barrier()

    idxb = (idx0, idx1)
    updb = (upd0, upd1)
    sems = (sm0, sm1)

    def stage(t, b):
        j = c * CH_PER_CORE + t * NS + s
        pltpu.async_copy(idx2_hbm.at[j], idxb[b], sems[b])
        pltpu.async_copy(upd_hbm.at[pl.ds(j * 128, 128)], updb[b], sems[b])

    def wait_stage(t, b):
        j = c * CH_PER_CORE + t * NS + s
        pltpu.make_async_copy(idx2_hbm.at[j], idxb[b], sems[b]).wait()
        pltpu.make_async_copy(
            upd_hbm.at[pl.ds(j * 128, 128)], updb[b], sems[b]).wait()

    stage(0, 0)

    def body(o, carry):
        t0 = o * 2
        for b in range(2):
            t = t0 + b

            @pl.when(t * NS + s < CH_PER_CORE)
            def _():
                wait_stage(t, b)

            @pl.when((t + 1) * NS + s < CH_PER_CORE)
            def _():
                stage(t + 1, 1 - b)

            @pl.when(t * NS + s < CH_PER_CORE)
            def _():
                pltpu.sync_copy(updb[b], acc_sh.at[idxb[b]], add=True)

        return carry

    lax.fori_loop(0, T_ITERS // 2, body, 0)
    plsc.subcore_barrier()

    @pl.when(s < 10)
    def _():
        pltpu.sync_copy(acc_sh.at[pl.ds(s * 1000, 1000)],
                        out_hbm.at[c, pl.ds(s * 1000, 1000)])


BE = 1280
GRID = EC // BE
SLAB = EC // BE


def _mlp_body(gi, gj, gk, geo, sf, wi, wj, wk, wg, b1, wr, br, out):
    f = jnp.float32
    h = (jnp.dot(gi[...].astype(jnp.bfloat16), wi[...], preferred_element_type=f)
         + jnp.dot(gj[...].astype(jnp.bfloat16), wj[...], preferred_element_type=f)
         + jnp.dot(gk[...].astype(jnp.bfloat16), wk[...], preferred_element_type=f)
         + jnp.dot(geo[...].astype(jnp.bfloat16), wg[...], preferred_element_type=f))
    h = jnp.maximum(h + b1[...][None, :], 0.0)
    for l in range(DEPTH):
        h = jnp.dot(h.astype(jnp.bfloat16), wr[l],
                    preferred_element_type=f)
        h = jnp.maximum(h + br[l][None, :], 0.0)
    out[...] = jnp.where(sf[...] > 0.0, h[:, :H], h[:, H:])


def _mlp_call(gath, geo, sf, wi, wj, wk, wg, b1, wrbd, brc):
    full = lambda a: pl.BlockSpec(a.shape, lambda e: (0,) * a.ndim)
    return pl.pallas_call(
        _mlp_body,
        grid=(GRID,),
        in_specs=[
            pl.BlockSpec((BE, H), lambda e: (e, 0)),
            pl.BlockSpec((BE, H), lambda e: (e + SLAB, 0)),
            pl.BlockSpec((BE, H), lambda e: (e + 2 * SLAB, 0)),
            pl.BlockSpec((BE, GEO), lambda e: (e, 0)),
            pl.BlockSpec((BE, 1), lambda e: (e, 0)),
            full(wi), full(wj), full(wk), full(wg), full(b1),
            full(wrbd), full(brc),
        ],
        out_specs=pl.BlockSpec((BE, H), lambda e: (e, 0)),
        out_shape=jax.ShapeDtypeStruct((EC, H), jnp.float32),
        compiler_params=pltpu.CompilerParams(
            dimension_semantics=("arbitrary",)),
    )(gath, gath, gath, geo, sf, wi, wj, wk, wg, b1, wrbd, brc)


def _add_body(p, out):
    out[...] = jnp.sum(p[...], axis=(0, 1))


def _add_call(partials):
    npart = partials.shape[0]
    return pl.pallas_call(
        _add_body,
        grid=(10,),
        in_specs=[pl.BlockSpec((npart, NC, N // 10, H),
                               lambda e: (0, 0, e, 0))],
        out_specs=pl.BlockSpec((N // 10, H), lambda e: (e, 0)),
        out_shape=jax.ShapeDtypeStruct((N, H), jnp.float32),
    )(partials)


def kernel(node_feature, geo_encoding, edge_index_2rd, edx_jk, edx_ij,
           edge_whichface, att, same_face, W_first, b_first, W_rest, b_rest):
    sf = same_face.astype(jnp.float32).reshape(E, 1)

    wf_t = jnp.transpose(W_first, (0, 2, 1))
    wr_t = jnp.transpose(W_rest, (0, 1, 3, 2))
    a = att[:, 0]
    wr_t = wr_t.at[:, DEPTH - 1].multiply(a[:, None, None])
    br = b_rest.at[:, DEPTH - 1].multiply(a[:, None])

    wcat = jnp.concatenate([wf_t[0], wf_t[1]], axis=1)
    wi = wcat[:H].astype(jnp.bfloat16)
    wj = wcat[H:2 * H].astype(jnp.bfloat16)
    wk = wcat[2 * H:3 * H].astype(jnp.bfloat16)
    wg = wcat[3 * H:].astype(jnp.bfloat16)
    b1 = jnp.concatenate([b_first[0], b_first[1]])
    z = jnp.zeros((DEPTH, H, H), jnp.float32)
    wrbd = jnp.concatenate([
        jnp.concatenate([wr_t[0], z], axis=2),
        jnp.concatenate([z, wr_t[1]], axis=2),
    ], axis=1).astype(jnp.bfloat16)
    brc = jnp.concatenate([br[0], br[1]], axis=1)

    ei = edge_index_2rd.astype(jnp.int32)
    zeros = jnp.zeros((N, H), jnp.float32)
    gfn = _gather_kernel_fn()
    sfn = _scatter_kernel_fn()

    gaths = []
    for cidx in range(NCHUNK):
        lo = cidx * EC
        idx_c = ei[:, lo:lo + EC].reshape(G_ROWS)
        gaths.append(gfn(node_feature, idx_c))
    eos = []
    for cidx in range(NCHUNK):
        lo = cidx * EC
        eos.append(_mlp_call(gaths[cidx], geo_encoding[lo:lo + EC],
                             sf[lo:lo + EC], wi, wj, wk, wg, b1, wrbd, brc))
    parts = []
    for cidx in range(NCHUNK):
        lo = cidx * EC
        idx2_c = ei[0, lo:lo + EC].reshape(CH, 128)
        parts.append(sfn(eos[cidx], idx2_c, zeros))

    return _add_call(jnp.stack(parts))


# speedup vs baseline: 3.6838x; 1.1166x over previous
import functools

import jax
import jax.numpy as jnp
from jax import lax
from jax.experimental import pallas as pl
from jax.experimental.pallas import tpu as pltpu
from jax.experimental.pallas import tpu_sc as plsc

N = 10000
E = 320000
H = 128
GEO = 16
DEPTH = 3

NC = 2
NS = 16
NW = NC * NS

NCHUNK = 5
EC = E // NCHUNK

G_ROWS = 3 * EC
R_PER_W = G_ROWS // NW
G_CHUNK = 120
G_ITERS = R_PER_W // G_CHUNK


@functools.cache
def _sc_mesh():
    return plsc.VectorSubcoreMesh(core_axis_name="c", subcore_axis_name="s",
                                  num_cores=NC, num_subcores=NS)


@functools.cache
def _gather_kernel_fn(cidx):
    del cidx
    return pl.kernel(
        _gather_body,
        out_type=jax.ShapeDtypeStruct((G_ROWS, H), jnp.float32),
        mesh=_sc_mesh(),
        scratch_types=[
            pltpu.VMEM((R_PER_W,), jnp.int32),
            pltpu.VMEM((G_CHUNK, H), jnp.float32),
            pltpu.VMEM((G_CHUNK, H), jnp.float32),
            pltpu.VMEM_SHARED((N, H), jnp.float32),
            pltpu.SemaphoreType.DMA,
            pltpu.SemaphoreType.DMA,
            pltpu.SemaphoreType.DMA,
            pltpu.SemaphoreType.DMA,
        ],
    )


def _gather_body(table_hbm, idx_hbm, out_hbm, idx_v, buf0, buf1, table_sh,
                 sg0, sg1, so0, so1):
    c = lax.axis_index("c")
    sid = lax.axis_index("s")
    wid = sid * NC + c
    base = wid * R_PER_W

    @pl.when(sid < 10)
    def _():
        pltpu.sync_copy(table_hbm.at[pl.ds(sid * 1000, 1000)],
                        table_sh.at[pl.ds(sid * 1000, 1000)])

    pltpu.sync_copy(idx_hbm.at[pl.ds(base, R_PER_W)], idx_v)
    plsc.subcore_barrier()

    bufs = (buf0, buf1)
    gsems = (sg0, sg1)
    osems = (so0, so1)

    def start_g(t, b):
        return pltpu.async_copy(
            table_sh.at[idx_v.at[pl.ds(t * G_CHUNK, G_CHUNK)]],
            bufs[b], gsems[b])

    def start_o(t, b):
        return pltpu.async_copy(
            bufs[b], out_hbm.at[pl.ds(base + t * G_CHUNK, G_CHUNK)], osems[b])

    start_g(0, 0)
    start_g(1, 1)

    def body(o, carry):
        t0 = o * 2
        for b in range(2):
            t = t0 + b
            pltpu.make_async_copy(
                table_sh.at[idx_v.at[pl.ds(t * G_CHUNK, G_CHUNK)]],
                bufs[b], gsems[b]).wait()
            start_o(t, b)
        for b in range(2):
            t = t0 + b
            pltpu.make_async_copy(
                bufs[b], out_hbm.at[pl.ds(base + t * G_CHUNK, G_CHUNK)],
                osems[b]).wait()

            @pl.when(t + 2 < G_ITERS)
            def _():
                start_g(t + 2, b)

        return carry

    lax.fori_loop(0, G_ITERS // 2, body, 0)


CH = EC // 128
CH_PER_CORE = CH // NC
T_ITERS = (CH_PER_CORE + NS - 1) // NS


@functools.cache
def _scatter_kernel_fn():
    return pl.kernel(
        _scatter_body,
        out_type=jax.ShapeDtypeStruct((NC, N, H), jnp.float32),
        mesh=_sc_mesh(),
        scratch_types=[
            pltpu.VMEM((128,), jnp.int32),
            pltpu.VMEM((128,), jnp.int32),
            pltpu.VMEM((128, H), jnp.float32),
            pltpu.VMEM((128, H), jnp.float32),
            pltpu.VMEM_SHARED((N, H), jnp.float32),
            pltpu.SemaphoreType.DMA,
            pltpu.SemaphoreType.DMA,
        ],
    )


def _scatter_body(upd_hbm, idx2_hbm, zeros_hbm, out_hbm,
                  idx0, idx1, upd0, upd1, acc_sh, sm0, sm1):
    c = lax.axis_index("c")
    s = lax.axis_index("s")

    @pl.when(s == 0)
    def _():
        pltpu.sync_copy(zeros_hbm, acc_sh)

    plsc.subcore_barrier()

    idxb = (idx0, idx1)
    updb = (upd0, upd1)
    sems = (sm0, sm1)

    def stage(t, b):
        j = c * CH_PER_CORE + t * NS + s
        pltpu.async_copy(idx2_hbm.at[j], idxb[b], sems[b])
        pltpu.async_copy(upd_hbm.at[pl.ds(j * 128, 128)], updb[b], sems[b])

    def wait_stage(t, b):
        j = c * CH_PER_CORE + t * NS + s
        pltpu.make_async_copy(idx2_hbm.at[j], idxb[b], sems[b]).wait()
        pltpu.make_async_copy(
            upd_hbm.at[pl.ds(j * 128, 128)], updb[b], sems[b]).wait()

    stage(0, 0)

    def body(o, carry):
        t0 = o * 2
        for b in range(2):
            t = t0 + b

            @pl.when(t * NS + s < CH_PER_CORE)
            def _():
                wait_stage(t, b)

            @pl.when((t + 1) * NS + s < CH_PER_CORE)
            def _():
                stage(t + 1, 1 - b)

            @pl.when(t * NS + s < CH_PER_CORE)
            def _():
                pltpu.sync_copy(updb[b], acc_sh.at[idxb[b]], add=True)

        return carry

    lax.fori_loop(0, T_ITERS // 2, body, 0)
    plsc.subcore_barrier()

    @pl.when(s < 10)
    def _():
        pltpu.sync_copy(acc_sh.at[pl.ds(s * 1000, 1000)],
                        out_hbm.at[c, pl.ds(s * 1000, 1000)])


BE = 1280
GRID = EC // BE
SLAB = EC // BE


def _mlp_body(gi, gj, gk, geo, sf, wi, wj, wk, wg, b1, wr, br, out):
    f = jnp.float32
    h = (jnp.dot(gi[...].astype(jnp.bfloat16), wi[...], preferred_element_type=f)
         + jnp.dot(gj[...].astype(jnp.bfloat16), wj[...], preferred_element_type=f)
         + jnp.dot(gk[...].astype(jnp.bfloat16), wk[...], preferred_element_type=f)
         + jnp.dot(geo[...].astype(jnp.bfloat16), wg[...], preferred_element_type=f))
    h = jnp.maximum(h + b1[...][None, :], 0.0)
    for l in range(DEPTH):
        h = jnp.dot(h.astype(jnp.bfloat16), wr[l],
                    preferred_element_type=f)
        h = jnp.maximum(h + br[l][None, :], 0.0)
    out[...] = jnp.where(sf[...] > 0.0, h[:, :H], h[:, H:])


def _mlp_call(gath, geo, sf, wi, wj, wk, wg, b1, wrbd, brc):
    full = lambda a: pl.BlockSpec(a.shape, lambda e: (0,) * a.ndim)
    return pl.pallas_call(
        _mlp_body,
        grid=(GRID,),
        in_specs=[
            pl.BlockSpec((BE, H), lambda e: (e, 0)),
            pl.BlockSpec((BE, H), lambda e: (e + SLAB, 0)),
            pl.BlockSpec((BE, H), lambda e: (e + 2 * SLAB, 0)),
            pl.BlockSpec((BE, GEO), lambda e: (e, 0)),
            pl.BlockSpec((BE, 1), lambda e: (e, 0)),
            full(wi), full(wj), full(wk), full(wg), full(b1),
            full(wrbd), full(brc),
        ],
        out_specs=pl.BlockSpec((BE, H), lambda e: (e, 0)),
        out_shape=jax.ShapeDtypeStruct((EC, H), jnp.float32),
        compiler_params=pltpu.CompilerParams(
            dimension_semantics=("arbitrary",)),
    )(gath, gath, gath, geo, sf, wi, wj, wk, wg, b1, wrbd, brc)


def _add_body(p, out):
    out[...] = jnp.sum(p[...], axis=(0, 1))


def _add_call(partials):
    npart = partials.shape[0]
    return pl.pallas_call(
        _add_body,
        grid=(10,),
        in_specs=[pl.BlockSpec((npart, NC, N // 10, H),
                               lambda e: (0, 0, e, 0))],
        out_specs=pl.BlockSpec((N // 10, H), lambda e: (e, 0)),
        out_shape=jax.ShapeDtypeStruct((N, H), jnp.float32),
    )(partials)


def kernel(node_feature, geo_encoding, edge_index_2rd, edx_jk, edx_ij,
           edge_whichface, att, same_face, W_first, b_first, W_rest, b_rest):
    sf = same_face.astype(jnp.float32).reshape(E, 1)

    wf_t = jnp.transpose(W_first, (0, 2, 1))
    wr_t = jnp.transpose(W_rest, (0, 1, 3, 2))
    a = att[:, 0]
    wr_t = wr_t.at[:, DEPTH - 1].multiply(a[:, None, None])
    br = b_rest.at[:, DEPTH - 1].multiply(a[:, None])

    wcat = jnp.concatenate([wf_t[0], wf_t[1]], axis=1)
    wi = wcat[:H].astype(jnp.bfloat16)
    wj = wcat[H:2 * H].astype(jnp.bfloat16)
    wk = wcat[2 * H:3 * H].astype(jnp.bfloat16)
    wg = wcat[3 * H:].astype(jnp.bfloat16)
    b1 = jnp.concatenate([b_first[0], b_first[1]])
    z = jnp.zeros((DEPTH, H, H), jnp.float32)
    wrbd = jnp.concatenate([
        jnp.concatenate([wr_t[0], z], axis=2),
        jnp.concatenate([z, wr_t[1]], axis=2),
    ], axis=1).astype(jnp.bfloat16)
    brc = jnp.concatenate([br[0], br[1]], axis=1)

    ei = edge_index_2rd.astype(jnp.int32)
    zeros = jnp.zeros((N, H), jnp.float32)
    sfn = _scatter_kernel_fn()

    gaths = []
    for cidx in range(NCHUNK):
        lo = cidx * EC
        idx_c = ei[:, lo:lo + EC].reshape(G_ROWS)
        gaths.append(_gather_kernel_fn(cidx)(node_feature, idx_c))
    eos = []
    for cidx in range(NCHUNK):
        lo = cidx * EC
        eos.append(_mlp_call(gaths[cidx], geo_encoding[lo:lo + EC],
                             sf[lo:lo + EC], wi, wj, wk, wg, b1, wrbd, brc))
    parts = []
    for cidx in range(NCHUNK):
        lo = cidx * EC
        idx2_c = ei[0, lo:lo + EC].reshape(CH, 128)
        parts.append(sfn(eos[cidx], idx2_c, zeros))

    return _add_call(jnp.stack(parts))


# speedup vs baseline: 3.8463x; 1.0441x over previous
import functools

import jax
import jax.numpy as jnp
from jax import lax
from jax.experimental import pallas as pl
from jax.experimental.pallas import tpu as pltpu
from jax.experimental.pallas import tpu_sc as plsc

N = 10000
E = 320000
H = 128
GEO = 16
DEPTH = 3

NC = 2
NS = 16
NW = NC * NS

NCHUNK = 2
EC = E // NCHUNK

G_ROWS = 3 * EC
R_PER_W = G_ROWS // NW
G_CHUNK = 120
G_ITERS = R_PER_W // G_CHUNK


@functools.cache
def _sc_mesh():
    return plsc.VectorSubcoreMesh(core_axis_name="c", subcore_axis_name="s",
                                  num_cores=NC, num_subcores=NS)


@functools.cache
def _gather_kernel_fn(cidx):
    del cidx
    return pl.kernel(
        _gather_body,
        out_type=jax.ShapeDtypeStruct((G_ROWS, H), jnp.float32),
        mesh=_sc_mesh(),
        scratch_types=[
            pltpu.VMEM((R_PER_W,), jnp.int32),
            pltpu.VMEM((G_CHUNK, H), jnp.float32),
            pltpu.VMEM((G_CHUNK, H), jnp.float32),
            pltpu.VMEM_SHARED((N, H), jnp.float32),
            pltpu.SemaphoreType.DMA,
            pltpu.SemaphoreType.DMA,
            pltpu.SemaphoreType.DMA,
            pltpu.SemaphoreType.DMA,
        ],
    )


def _gather_body(table_hbm, idx_hbm, out_hbm, idx_v, buf0, buf1, table_sh,
                 sg0, sg1, so0, so1):
    c = lax.axis_index("c")
    sid = lax.axis_index("s")
    wid = sid * NC + c
    base = wid * R_PER_W

    @pl.when(sid < 10)
    def _():
        pltpu.sync_copy(table_hbm.at[pl.ds(sid * 1000, 1000)],
                        table_sh.at[pl.ds(sid * 1000, 1000)])

    pltpu.sync_copy(idx_hbm.at[pl.ds(base, R_PER_W)], idx_v)
    plsc.subcore_barrier()

    bufs = (buf0, buf1)
    gsems = (sg0, sg1)
    osems = (so0, so1)

    def start_g(t, b):
        return pltpu.async_copy(
            table_sh.at[idx_v.at[pl.ds(t * G_CHUNK, G_CHUNK)]],
            bufs[b], gsems[b])

    def start_o(t, b):
        return pltpu.async_copy(
            bufs[b], out_hbm.at[pl.ds(base + t * G_CHUNK, G_CHUNK)], osems[b])

    start_g(0, 0)
    start_g(1, 1)

    def body(o, carry):
        t0 = o * 2
        for b in range(2):
            t = t0 + b
            pltpu.make_async_copy(
                table_sh.at[idx_v.at[pl.ds(t * G_CHUNK, G_CHUNK)]],
                bufs[b], gsems[b]).wait()
            start_o(t, b)
        for b in range(2):
            t = t0 + b
            pltpu.make_async_copy(
                bufs[b], out_hbm.at[pl.ds(base + t * G_CHUNK, G_CHUNK)],
                osems[b]).wait()

            @pl.when(t + 2 < G_ITERS)
            def _():
                start_g(t + 2, b)

        return carry

    lax.fori_loop(0, G_ITERS // 2, body, 0)

    t_last = G_ITERS - 1
    pltpu.make_async_copy(
        table_sh.at[idx_v.at[pl.ds(t_last * G_CHUNK, G_CHUNK)]],
        bufs[0], gsems[0]).wait()
    start_o(t_last, 0)
    pltpu.make_async_copy(
        bufs[0], out_hbm.at[pl.ds(base + t_last * G_CHUNK, G_CHUNK)],
        osems[0]).wait()


CH = EC // 128
CH_PER_CORE = CH // NC
T_ITERS = 40


@functools.cache
def _scatter_kernel_fn():
    return pl.kernel(
        _scatter_body,
        out_type=jax.ShapeDtypeStruct((NC, N, H), jnp.float32),
        mesh=_sc_mesh(),
        scratch_types=[
            pltpu.VMEM((128,), jnp.int32),
            pltpu.VMEM((128,), jnp.int32),
            pltpu.VMEM((128, H), jnp.float32),
            pltpu.VMEM((128, H), jnp.float32),
            pltpu.VMEM_SHARED((N, H), jnp.float32),
            pltpu.SemaphoreType.DMA,
            pltpu.SemaphoreType.DMA,
        ],
    )


def _scatter_body(upd_hbm, idx2_hbm, zeros_hbm, out_hbm,
                  idx0, idx1, upd0, upd1, acc_sh, sm0, sm1):
    c = lax.axis_index("c")
    s = lax.axis_index("s")

    @pl.when(s == 0)
    def _():
        pltpu.sync_copy(zeros_hbm, acc_sh)

    plsc.subcore_barrier()

    idxb = (idx0, idx1)
    updb = (upd0, upd1)
    sems = (sm0, sm1)

    def stage(t, b):
        j = c * CH_PER_CORE + t * NS + s
        pltpu.async_copy(idx2_hbm.at[j], idxb[b], sems[b])
        pltpu.async_copy(upd_hbm.at[pl.ds(j * 128, 128)], updb[b], sems[b])

    def wait_stage(t, b):
        j = c * CH_PER_CORE + t * NS + s
        pltpu.make_async_copy(idx2_hbm.at[j], idxb[b], sems[b]).wait()
        pltpu.make_async_copy(
            upd_hbm.at[pl.ds(j * 128, 128)], updb[b], sems[b]).wait()

    stage(0, 0)

    def body(o, carry):
        t0 = o * 2
        for b in range(2):
            t = t0 + b

            @pl.when(t * NS + s < CH_PER_CORE)
            def _():
                wait_stage(t, b)

            @pl.when((t + 1) * NS + s < CH_PER_CORE)
            def _():
                stage(t + 1, 1 - b)

            @pl.when(t * NS + s < CH_PER_CORE)
            def _():
                pltpu.sync_copy(updb[b], acc_sh.at[idxb[b]], add=True)

        return carry

    lax.fori_loop(0, T_ITERS // 2, body, 0)
    plsc.subcore_barrier()

    @pl.when(s < 10)
    def _():
        pltpu.sync_copy(acc_sh.at[pl.ds(s * 1000, 1000)],
                        out_hbm.at[c, pl.ds(s * 1000, 1000)])


BE = 3200
GRID = EC // BE
SLAB = EC // BE


def _mlp_body(gi, gj, gk, geo, sf, wi, wj, wk, wg, b1, wr, br, out):
    f = jnp.float32
    h = (jnp.dot(gi[...].astype(jnp.bfloat16), wi[...], preferred_element_type=f)
         + jnp.dot(gj[...].astype(jnp.bfloat16), wj[...], preferred_element_type=f)
         + jnp.dot(gk[...].astype(jnp.bfloat16), wk[...], preferred_element_type=f)
         + jnp.dot(geo[...].astype(jnp.bfloat16), wg[...], preferred_element_type=f))
    h = jnp.maximum(h + b1[...][None, :], 0.0)
    for l in range(DEPTH):
        h = jnp.dot(h.astype(jnp.bfloat16), wr[l],
                    preferred_element_type=f)
        h = jnp.maximum(h + br[l][None, :], 0.0)
    out[...] = jnp.where(sf[...] > 0.0, h[:, :H], h[:, H:])


def _mlp_call(gath, geo, sf, wi, wj, wk, wg, b1, wrbd, brc):
    full = lambda a: pl.BlockSpec(a.shape, lambda e: (0,) * a.ndim)
    return pl.pallas_call(
        _mlp_body,
        grid=(GRID,),
        in_specs=[
            pl.BlockSpec((BE, H), lambda e: (e, 0)),
            pl.BlockSpec((BE, H), lambda e: (e + SLAB, 0)),
            pl.BlockSpec((BE, H), lambda e: (e + 2 * SLAB, 0)),
            pl.BlockSpec((BE, GEO), lambda e: (e, 0)),
            pl.BlockSpec((BE, 1), lambda e: (e, 0)),
            full(wi), full(wj), full(wk), full(wg), full(b1),
            full(wrbd), full(brc),
        ],
        out_specs=pl.BlockSpec((BE, H), lambda e: (e, 0)),
        out_shape=jax.ShapeDtypeStruct((EC, H), jnp.float32),
        compiler_params=pltpu.CompilerParams(
            dimension_semantics=("arbitrary",)),
    )(gath, gath, gath, geo, sf, wi, wj, wk, wg, b1, wrbd, brc)


def _add_body(p, out):
    out[...] = jnp.sum(p[...], axis=(0, 1))


def _add_call(partials):
    npart = partials.shape[0]
    return pl.pallas_call(
        _add_body,
        grid=(10,),
        in_specs=[pl.BlockSpec((npart, NC, N // 10, H),
                               lambda e: (0, 0, e, 0))],
        out_specs=pl.BlockSpec((N // 10, H), lambda e: (e, 0)),
        out_shape=jax.ShapeDtypeStruct((N, H), jnp.float32),
    )(partials)


def kernel(node_feature, geo_encoding, edge_index_2rd, edx_jk, edx_ij,
           edge_whichface, att, same_face, W_first, b_first, W_rest, b_rest):
    sf = same_face.astype(jnp.float32).reshape(E, 1)

    wf_t = jnp.transpose(W_first, (0, 2, 1))
    wr_t = jnp.transpose(W_rest, (0, 1, 3, 2))
    a = att[:, 0]
    wr_t = wr_t.at[:, DEPTH - 1].multiply(a[:, None, None])
    br = b_rest.at[:, DEPTH - 1].multiply(a[:, None])

    wcat = jnp.concatenate([wf_t[0], wf_t[1]], axis=1)
    wi = wcat[:H].astype(jnp.bfloat16)
    wj = wcat[H:2 * H].astype(jnp.bfloat16)
    wk = wcat[2 * H:3 * H].astype(jnp.bfloat16)
    wg = wcat[3 * H:].astype(jnp.bfloat16)
    b1 = jnp.concatenate([b_first[0], b_first[1]])
    z = jnp.zeros((DEPTH, H, H), jnp.float32)
    wrbd = jnp.concatenate([
        jnp.concatenate([wr_t[0], z], axis=2),
        jnp.concatenate([z, wr_t[1]], axis=2),
    ], axis=1).astype(jnp.bfloat16)
    brc = jnp.concatenate([br[0], br[1]], axis=1)

    ei = edge_index_2rd.astype(jnp.int32)
    zeros = jnp.zeros((N, H), jnp.float32)
    sfn = _scatter_kernel_fn()

    gaths = []
    for cidx in range(NCHUNK):
        lo = cidx * EC
        idx_c = ei[:, lo:lo + EC].reshape(G_ROWS)
        gaths.append(_gather_kernel_fn(cidx)(node_feature, idx_c))
    eos = []
    for cidx in range(NCHUNK):
        lo = cidx * EC
        eos.append(_mlp_call(gaths[cidx], geo_encoding[lo:lo + EC],
                             sf[lo:lo + EC], wi, wj, wk, wg, b1, wrbd, brc))
    parts = []
    for cidx in range(NCHUNK):
        lo = cidx * EC
        idx2_c = ei[0, lo:lo + EC].reshape(CH, 128)
        parts.append(sfn(eos[cidx], idx2_c, zeros))

    return _add_call(jnp.stack(parts))
